# baseline (speedup 1.0000x reference)
import functools

import jax
import jax.numpy as jnp
from jax import lax
from jax.experimental import pallas as pl
from jax.experimental.pallas import tpu as pltpu
from jax.experimental.pallas import tpu_sc as plsc

N = 10000
NP = 10240
E = 320000
NC, NS, L = 2, 16, 16
NW = NC * NS
K = 128
CHUNKS = 80
EPW = CHUNKS * K
E_PAD = NW * EPW
PAIRS = CHUNKS // 2

F32 = jnp.float32


def _make_norm():
  mesh = plsc.VectorSubcoreMesh(core_axis_name="c", subcore_axis_name="s")

  @functools.partial(
      pl.kernel,
      out_type=jax.ShapeDtypeStruct((NW * CHUNKS, K), F32),
      mesh=mesh,
      compiler_params=pltpu.CompilerParams(use_tc_tiling_on_sc=False,
                                           needs_layout_passes=False),
      scratch_types=[
          pltpu.VMEM((CHUNKS, K), jnp.int32),
          pltpu.VMEM((CHUNKS, K), jnp.int32),
          pltpu.VMEM((CHUNKS, K), F32),
          pltpu.VMEM((N,), F32),
      ],
  )
  def normk(src_hbm, dst_hbm, ew_hbm, dinv_hbm, out_hbm,
            src_v, dst_v, ew_v, dinv_v):
    c = lax.axis_index("c")
    s = lax.axis_index("s")
    base = (c * NS + s) * CHUNKS
    pltpu.sync_copy(src_hbm.at[pl.ds(base, CHUNKS)], src_v)
    pltpu.sync_copy(dst_hbm.at[pl.ds(base, CHUNKS)], dst_v)
    pltpu.sync_copy(ew_hbm.at[pl.ds(base, CHUNKS)], ew_v)
    pltpu.sync_copy(dinv_hbm, dinv_v)

    def nchunk(g, _):
      for j in range(K // L):
        sl = pl.ds(j * L, L)
        s16 = src_v[g, sl]
        d16 = dst_v[g, sl]
        w16 = ew_v[g, sl]
        nv = plsc.load_gather(dinv_v, [s16]) * w16
        ew_v[g, sl] = nv * plsc.load_gather(dinv_v, [d16])
      return 0
    lax.fori_loop(0, CHUNKS, nchunk, 0)
    pltpu.sync_copy(ew_v, out_hbm.at[pl.ds(base, CHUNKS)])

  return normk


def _make_agg(w):
  rows_per_sub = NP // NS
  zcopies = rows_per_sub // K
  mesh = plsc.VectorSubcoreMesh(core_axis_name="c", subcore_axis_name="s")

  @functools.partial(
      pl.kernel,
      out_type=jax.ShapeDtypeStruct((NC, NP, w), F32),
      mesh=mesh,
      compiler_params=pltpu.CompilerParams(use_tc_tiling_on_sc=False,
                                           needs_layout_passes=False),
      scratch_types=[
          pltpu.VMEM((CHUNKS, K), jnp.int32),
          pltpu.VMEM((CHUNKS, K), jnp.int32),
          pltpu.VMEM((CHUNKS, K), F32),
          pltpu.VMEM((K, w), F32),
          pltpu.VMEM((K, w), F32),
          pltpu.VMEM_SHARED((NP, w), F32),
          pltpu.SemaphoreType.DMA,
          pltpu.SemaphoreType.DMA,
          pltpu.SemaphoreType.DMA,
          pltpu.SemaphoreType.DMA,
      ],
  )
  def agg(g_hbm, src_hbm, dst_hbm, norm_hbm, out_hbm,
          src_v, dst_v, norm_v, rows_a, rows_b, acc_sh,
          sem_ga, sem_gb, sem_sa, sem_sb):
    c = lax.axis_index("c")
    s = lax.axis_index("s")
    base = (c * NS + s) * CHUNKS

    pltpu.sync_copy(src_hbm.at[pl.ds(base, CHUNKS)], src_v)
    pltpu.sync_copy(dst_hbm.at[pl.ds(base, CHUNKS)], dst_v)
    pltpu.sync_copy(norm_hbm.at[pl.ds(base, CHUNKS)], norm_v)

    def zrow(i, _):
      for j in range(w // L):
        rows_a[i, pl.ds(j * L, L)] = jnp.zeros((L,), F32)
        rows_b[i, pl.ds(j * L, L)] = jnp.zeros((L,), F32)
      return 0
    lax.fori_loop(0, K, zrow, 0)
    for i in range(zcopies):
      pltpu.sync_copy(rows_a, acc_sh.at[pl.ds(s * rows_per_sub + i * K, K)])
    plsc.subcore_barrier()

    def g_start(g, rows, sem):
      pltpu.async_copy(g_hbm.at[src_v.at[g]], rows, sem)

    def g_wait(g, rows, sem):
      pltpu.make_async_copy(g_hbm.at[src_v.at[g]], rows, sem).wait()

    def s_start(g, rows, sem):
      pltpu.async_copy(rows, acc_sh.at[dst_v.at[g]], sem, add=True)

    def s_wait(g, rows, sem):
      pltpu.make_async_copy(rows, acc_sh.at[dst_v.at[g]], sem).wait()

    def scale(g, rows):
      def grp(j, _):
        nv = norm_v[g, pl.ds(j * L, L)]
        for t in range(L):
          nt = nv[t]
          e = j * L + t
          for jj in range(w // L):
            sl = pl.ds(jj * L, L)
            rows[e, sl] = rows[e, sl] * nt
        return 0
      lax.fori_loop(0, K // L, grp, 0)

    g_start(0, rows_a, sem_ga)
    s_start(0, rows_b, sem_sb)

    def pair(gi, _):
      ga = gi * 2
      gb = ga + 1
      s_wait(gb, rows_b, sem_sb)
      g_start(gb, rows_b, sem_gb)
      g_wait(ga, rows_a, sem_ga)
      scale(ga, rows_a)
      s_start(ga, rows_a, sem_sa)
      g_wait(gb, rows_b, sem_gb)
      scale(gb, rows_b)
      s_wait(ga, rows_a, sem_sa)
      g_start(jnp.minimum(ga + 2, CHUNKS - 1), rows_a, sem_ga)
      s_start(gb, rows_b, sem_sb)
      return 0
    lax.fori_loop(0, PAIRS, pair, 0)

    g_wait(CHUNKS - 1, rows_a, sem_ga)
    s_wait(CHUNKS - 1, rows_b, sem_sb)

    plsc.subcore_barrier()
    for i in range(zcopies):
      sl = pl.ds(s * rows_per_sub + i * K, K)
      pltpu.sync_copy(acc_sh.at[sl], out_hbm.at[c, sl])

  return agg


def _bn(h, g, b):
  mu = jnp.mean(h, axis=0, keepdims=True)
  var = jnp.var(h, axis=0, keepdims=True)
  return g * (h - mu) / jnp.sqrt(var + 1e-5) + b


def _tc(fn, out_shape, *args):
  return pl.pallas_call(fn, out_shape=out_shape)(*args)


DIMS = [(128, 128), (128, 128), (128, 64), (64, 32), (32, 64), (64, 128),
        (128, 64), (64, 32)]
AGG_W = [do for _, do in DIMS]


def _agg_sum(aggs, a_parts, srcp, dstp, normp):
  outs = []
  for ap in a_parts:
    s = aggs[ap.shape[1]](ap, srcp, dstp, normp)
    outs.append(s)
  return outs


def kernel(x, edge_index, edge_weight, params):
  p = params
  src = edge_index[0]
  dst = edge_index[1]
  pad = E_PAD - E
  srcp = jnp.pad(src, (0, pad)).reshape(NW * CHUNKS, K)
  dstp = jnp.pad(dst, (0, pad)).reshape(NW * CHUNKS, K)
  ewp = jnp.pad(edge_weight, (0, pad)).reshape(NW * CHUNKS, K)

  aggs = {w: _make_agg(w) for w in (16, 32, 64)}
  normk = _make_norm()

  ones16 = jnp.ones((N, 16), F32)
  s0 = aggs[16](ones16, srcp, dstp, ewp)

  def tc0(x_ref, w1_ref, g_ref, be_ref, s0_ref, dinv_ref, dinv2_ref,
          alo_ref, ahi_ref):
    deg = (s0_ref[0, :N, 0:1] + s0_ref[1, :N, 0:1]) + 1.0
    dinv = lax.rsqrt(deg)
    dinv_ref[...] = dinv
    dinv2_ref[...] = dinv * dinv
    h = _bn(x_ref[...], g_ref[...], be_ref[...])
    a1 = jnp.dot(h, w1_ref[...], preferred_element_type=F32)
    alo_ref[...] = a1[:, :64]
    ahi_ref[...] = a1[:, 64:]

  dinv, dinv2, a_lo, a_hi = _tc(
      tc0,
      (jax.ShapeDtypeStruct((N, 1), F32),
       jax.ShapeDtypeStruct((N, 1), F32),
       jax.ShapeDtypeStruct((N, 64), F32),
       jax.ShapeDtypeStruct((N, 64), F32)),
      x, p["W1"], p["g1"], p["be1"], s0)

  normp = normk(srcp, dstp, ewp, dinv.reshape((N,)))
  a_parts = [a_lo, a_hi]

  for i in range(1, 9):
    s_parts = _agg_sum(aggs, a_parts, srcp, dstp, normp)

    if i < 8:
      wn = AGG_W[i]
      split_n = wn == 128
      out_shapes = (
          (jax.ShapeDtypeStruct((N, 64), F32),
           jax.ShapeDtypeStruct((N, 64), F32)) if split_n
          else jax.ShapeDtypeStruct((N, wn), F32))

      def tci(*refs, nsp=len(s_parts), split_n=split_n):
        s_refs = refs[:nsp]
        a_refs = refs[nsp:2 * nsp]
        dinv2_ref, b_ref, g_ref, be_ref, wn_ref = refs[2 * nsp:2 * nsp + 5]
        out_refs = refs[2 * nsp + 5:]
        s_cat = jnp.concatenate(
            [sr[0, :N, :] + sr[1, :N, :] for sr in s_refs], axis=1)
        a_cat = jnp.concatenate([ar[...] for ar in a_refs], axis=1)
        conv = s_cat + a_cat * dinv2_ref[...]
        r = conv + b_ref[...]
        h = jax.nn.relu(_bn(r, g_ref[...], be_ref[...]))
        an = jnp.dot(h, wn_ref[...], preferred_element_type=F32)
        if split_n:
          out_refs[0][...] = an[:, :64]
          out_refs[1][...] = an[:, 64:]
        else:
          out_refs[0][...] = an

      res = _tc(
          tci, out_shapes,
          *s_parts, *a_parts, dinv2, p["b%d" % i],
          p["g%d" % (i + 1)], p["be%d" % (i + 1)], p["W%d" % (i + 1)])
      a_parts = list(res) if split_n else [res]
    else:
      def tc8(*refs, nsp=len(s_parts)):
        s_refs = refs[:nsp]
        a_refs = refs[nsp:2 * nsp]
        (dinv2_ref, b_ref, g9_ref, be9_ref, lw1_ref, lb1_ref,
         g10_ref, be10_ref, lw2_ref, lb2_ref, out_ref) = refs[2 * nsp:]
        s_cat = jnp.concatenate(
            [sr[0, :N, :] + sr[1, :N, :] for sr in s_refs], axis=1)
        a_cat = jnp.concatenate([ar[...] for ar in a_refs], axis=1)
        conv = s_cat + a_cat * dinv2_ref[...]
        h = _bn(conv + b_ref[...], g9_ref[...], be9_ref[...])
        t1 = jnp.dot(jax.nn.relu(h), lw1_ref[...],
                     preferred_element_type=F32) + lb1_ref[...]
        t1 = _bn(t1, g10_ref[...], be10_ref[...])
        out_ref[...] = jnp.dot(jax.nn.relu(t1), lw2_ref[...],
                               preferred_element_type=F32) + lb2_ref[...]

      out = _tc(
          tc8, jax.ShapeDtypeStruct((N, 40), F32),
          *s_parts, *a_parts, dinv2, p["b8"], p["g9"], p["be9"],
          p["lw1"], p["lb1"], p["g10"], p["be10"], p["lw2"], p["lb2"])

  return out


# speedup vs baseline: 1.2026x; 1.2026x over previous
import functools

import jax
import jax.numpy as jnp
from jax import lax
from jax.experimental import pallas as pl
from jax.experimental.pallas import tpu as pltpu
from jax.experimental.pallas import tpu_sc as plsc

N = 10000
NP = 10240
E = 320000
NC, NS, L = 2, 16, 16
NW = NC * NS
K = 128
CHUNKS = 80
EPW = CHUNKS * K
E_PAD = NW * EPW
PAIRS = CHUNKS // 2

F32 = jnp.float32


def _make_norm():
  mesh = plsc.VectorSubcoreMesh(core_axis_name="c", subcore_axis_name="s")

  @functools.partial(
      pl.kernel,
      out_type=jax.ShapeDtypeStruct((NW * CHUNKS, K), F32),
      mesh=mesh,
      compiler_params=pltpu.CompilerParams(use_tc_tiling_on_sc=False,
                                           needs_layout_passes=False),
      scratch_types=[
          pltpu.VMEM((CHUNKS, K), jnp.int32),
          pltpu.VMEM((CHUNKS, K), jnp.int32),
          pltpu.VMEM((CHUNKS, K), F32),
          pltpu.VMEM((N,), F32),
      ],
  )
  def normk(src_hbm, dst_hbm, ew_hbm, dinv_hbm, out_hbm,
            src_v, dst_v, ew_v, dinv_v):
    c = lax.axis_index("c")
    s = lax.axis_index("s")
    base = (c * NS + s) * CHUNKS
    pltpu.sync_copy(src_hbm.at[pl.ds(base, CHUNKS)], src_v)
    pltpu.sync_copy(dst_hbm.at[pl.ds(base, CHUNKS)], dst_v)
    pltpu.sync_copy(ew_hbm.at[pl.ds(base, CHUNKS)], ew_v)
    pltpu.sync_copy(dinv_hbm, dinv_v)

    def nchunk(g, _):
      for j in range(K // L):
        sl = pl.ds(j * L, L)
        s16 = src_v[g, sl]
        d16 = dst_v[g, sl]
        w16 = ew_v[g, sl]
        nv = plsc.load_gather(dinv_v, [s16]) * w16
        ew_v[g, sl] = nv * plsc.load_gather(dinv_v, [d16])
      return 0
    lax.fori_loop(0, CHUNKS, nchunk, 0)
    pltpu.sync_copy(ew_v, out_hbm.at[pl.ds(base, CHUNKS)])

  return normk


def _make_agg(w):
  rows_per_sub = NP // NS
  zcopies = rows_per_sub // K
  mesh = plsc.VectorSubcoreMesh(core_axis_name="c", subcore_axis_name="s")

  @functools.partial(
      pl.kernel,
      out_type=jax.ShapeDtypeStruct((NC, NP, w), F32),
      mesh=mesh,
      compiler_params=pltpu.CompilerParams(use_tc_tiling_on_sc=False,
                                           needs_layout_passes=False),
      scratch_types=[
          pltpu.VMEM((CHUNKS, K), jnp.int32),
          pltpu.VMEM((CHUNKS, K), jnp.int32),
          pltpu.VMEM((CHUNKS, K), F32),
          pltpu.VMEM((K, w), F32),
          pltpu.VMEM((K, w), F32),
          pltpu.VMEM_SHARED((NP, w), F32),
          pltpu.SemaphoreType.DMA,
          pltpu.SemaphoreType.DMA,
          pltpu.SemaphoreType.DMA,
          pltpu.SemaphoreType.DMA,
      ],
  )
  def agg(g_hbm, src_hbm, dst_hbm, norm_hbm, out_hbm,
          src_v, dst_v, norm_v, rows_a, rows_b, acc_sh,
          sem_ga, sem_gb, sem_sa, sem_sb):
    c = lax.axis_index("c")
    s = lax.axis_index("s")
    base = (c * NS + s) * CHUNKS

    def g_start(g, rows, sem):
      pltpu.async_copy(g_hbm.at[src_v.at[g]], rows, sem)

    def g_wait(g, rows, sem):
      pltpu.make_async_copy(g_hbm.at[src_v.at[g]], rows, sem).wait()

    def s_start(g, rows, sem):
      pltpu.async_copy(rows, acc_sh.at[dst_v.at[g]], sem, add=True)

    def s_wait(g, rows, sem):
      pltpu.make_async_copy(rows, acc_sh.at[dst_v.at[g]], sem).wait()

    pltpu.sync_copy(src_hbm.at[pl.ds(base, CHUNKS)], src_v)
    g_start(0, rows_a, sem_ga)
    pltpu.sync_copy(dst_hbm.at[pl.ds(base, CHUNKS)], dst_v)
    pltpu.sync_copy(norm_hbm.at[pl.ds(base, CHUNKS)], norm_v)

    def zrow(i, _):
      for j in range(w // L):
        rows_b[i, pl.ds(j * L, L)] = jnp.zeros((L,), F32)
      return 0
    lax.fori_loop(0, K, zrow, 0)
    for i in range(zcopies):
      pltpu.async_copy(rows_b, acc_sh.at[pl.ds(s * rows_per_sub + i * K, K)],
                       sem_sa)
    for i in range(zcopies):
      pltpu.make_async_copy(
          rows_b, acc_sh.at[pl.ds(s * rows_per_sub + i * K, K)],
          sem_sa).wait()
    plsc.subcore_barrier()

    def scale(g, rows):
      def grp(j, _):
        nv = norm_v[g, pl.ds(j * L, L)]
        for t in range(L):
          nt = nv[t]
          e = j * L + t
          for jj in range(w // L):
            sl = pl.ds(jj * L, L)
            rows[e, sl] = rows[e, sl] * nt
        return 0
      lax.fori_loop(0, K // L, grp, 0, unroll=2)

    s_start(0, rows_b, sem_sb)

    def pair(gi, _):
      ga = gi * 2
      gb = ga + 1
      s_wait(gb, rows_b, sem_sb)
      g_start(gb, rows_b, sem_gb)
      g_wait(ga, rows_a, sem_ga)
      scale(ga, rows_a)
      s_start(ga, rows_a, sem_sa)
      g_wait(gb, rows_b, sem_gb)
      scale(gb, rows_b)
      s_wait(ga, rows_a, sem_sa)
      g_start(jnp.minimum(ga + 2, CHUNKS - 1), rows_a, sem_ga)
      s_start(gb, rows_b, sem_sb)
      return 0
    lax.fori_loop(0, PAIRS, pair, 0)

    g_wait(CHUNKS - 1, rows_a, sem_ga)
    s_wait(CHUNKS - 1, rows_b, sem_sb)

    plsc.subcore_barrier()
    for i in range(zcopies):
      sl = pl.ds(s * rows_per_sub + i * K, K)
      pltpu.async_copy(acc_sh.at[sl], out_hbm.at[c, sl], sem_sa)
    for i in range(zcopies):
      sl = pl.ds(s * rows_per_sub + i * K, K)
      pltpu.make_async_copy(acc_sh.at[sl], out_hbm.at[c, sl], sem_sa).wait()

  return agg


def _bn(h, g, b):
  mu = jnp.mean(h, axis=0, keepdims=True)
  var = jnp.var(h, axis=0, keepdims=True)
  return g * (h - mu) / jnp.sqrt(var + 1e-5) + b


def _tc(fn, out_shape, *args):
  return pl.pallas_call(fn, out_shape=out_shape)(*args)


DIMS = [(128, 128), (128, 128), (128, 64), (64, 32), (32, 64), (64, 128),
        (128, 64), (64, 32)]
AGG_W = [do for _, do in DIMS]


def _agg_sum(aggs, a_parts, srcp, dstp, normp):
  outs = []
  for ap in a_parts:
    s = aggs[ap.shape[1]](ap, srcp, dstp, normp)
    outs.append(s)
  return outs


def kernel(x, edge_index, edge_weight, params):
  p = params
  src = edge_index[0]
  dst = edge_index[1]
  pad = E_PAD - E
  srcp = jnp.pad(src, (0, pad)).reshape(NW * CHUNKS, K)
  dstp = jnp.pad(dst, (0, pad)).reshape(NW * CHUNKS, K)
  ewp = jnp.pad(edge_weight, (0, pad)).reshape(NW * CHUNKS, K)

  aggs = {w: _make_agg(w) for w in (16, 32, 64)}
  normk = _make_norm()

  ones16 = jnp.ones((N, 16), F32)
  s0 = aggs[16](ones16, srcp, dstp, ewp)

  def tc0(x_ref, w1_ref, g_ref, be_ref, s0_ref, dinv_ref, dinv2_ref,
          alo_ref, ahi_ref):
    deg = (s0_ref[0, :N, 0:1] + s0_ref[1, :N, 0:1]) + 1.0
    dinv = lax.rsqrt(deg)
    dinv_ref[...] = dinv
    dinv2_ref[...] = dinv * dinv
    h = _bn(x_ref[...], g_ref[...], be_ref[...])
    a1 = jnp.dot(h, w1_ref[...], preferred_element_type=F32)
    alo_ref[...] = a1[:, :64]
    ahi_ref[...] = a1[:, 64:]

  dinv, dinv2, a_lo, a_hi = _tc(
      tc0,
      (jax.ShapeDtypeStruct((N, 1), F32),
       jax.ShapeDtypeStruct((N, 1), F32),
       jax.ShapeDtypeStruct((N, 64), F32),
       jax.ShapeDtypeStruct((N, 64), F32)),
      x, p["W1"], p["g1"], p["be1"], s0)

  normp = normk(srcp, dstp, ewp, dinv.reshape((N,)))
  a_parts = [a_lo, a_hi]

  for i in range(1, 9):
    s_parts = _agg_sum(aggs, a_parts, srcp, dstp, normp)

    if i < 8:
      wn = AGG_W[i]
      split_n = wn == 128
      out_shapes = (
          (jax.ShapeDtypeStruct((N, 64), F32),
           jax.ShapeDtypeStruct((N, 64), F32)) if split_n
          else jax.ShapeDtypeStruct((N, wn), F32))

      def tci(*refs, nsp=len(s_parts), split_n=split_n):
        s_refs = refs[:nsp]
        a_refs = refs[nsp:2 * nsp]
        dinv2_ref, b_ref, g_ref, be_ref, wn_ref = refs[2 * nsp:2 * nsp + 5]
        out_refs = refs[2 * nsp + 5:]
        s_cat = jnp.concatenate(
            [sr[0, :N, :] + sr[1, :N, :] for sr in s_refs], axis=1)
        a_cat = jnp.concatenate([ar[...] for ar in a_refs], axis=1)
        conv = s_cat + a_cat * dinv2_ref[...]
        r = conv + b_ref[...]
        h = jax.nn.relu(_bn(r, g_ref[...], be_ref[...]))
        an = jnp.dot(h, wn_ref[...], preferred_element_type=F32)
        if split_n:
          out_refs[0][...] = an[:, :64]
          out_refs[1][...] = an[:, 64:]
        else:
          out_refs[0][...] = an

      res = _tc(
          tci, out_shapes,
          *s_parts, *a_parts, dinv2, p["b%d" % i],
          p["g%d" % (i + 1)], p["be%d" % (i + 1)], p["W%d" % (i + 1)])
      a_parts = list(res) if split_n else [res]
    else:
      def tc8(*refs, nsp=len(s_parts)):
        s_refs = refs[:nsp]
        a_refs = refs[nsp:2 * nsp]
        (dinv2_ref, b_ref, g9_ref, be9_ref, lw1_ref, lb1_ref,
         g10_ref, be10_ref, lw2_ref, lb2_ref, out_ref) = refs[2 * nsp:]
        s_cat = jnp.concatenate(
            [sr[0, :N, :] + sr[1, :N, :] for sr in s_refs], axis=1)
        a_cat = jnp.concatenate([ar[...] for ar in a_refs], axis=1)
        conv = s_cat + a_cat * dinv2_ref[...]
        h = _bn(conv + b_ref[...], g9_ref[...], be9_ref[...])
        t1 = jnp.dot(jax.nn.relu(h), lw1_ref[...],
                     preferred_element_type=F32) + lb1_ref[...]
        t1 = _bn(t1, g10_ref[...], be10_ref[...])
        out_ref[...] = jnp.dot(jax.nn.relu(t1), lw2_ref[...],
                               preferred_element_type=F32) + lb2_ref[...]

      out = _tc(
          tc8, jax.ShapeDtypeStruct((N, 40), F32),
          *s_parts, *a_parts, dinv2, p["b8"], p["g9"], p["be9"],
          p["lw1"], p["lb1"], p["g10"], p["be10"], p["lw2"], p["lb2"])

  return out


# speedup vs baseline: 1.4923x; 1.2409x over previous
import functools

import jax
import jax.numpy as jnp
from jax import lax
from jax.experimental import pallas as pl
from jax.experimental.pallas import tpu as pltpu
from jax.experimental.pallas import tpu_sc as plsc

N = 10000
NP = 10240
E = 320000
NC, NS, L = 2, 16, 16
NW = NC * NS
K = 128
CHUNKS = 80
EPW = CHUNKS * K
E_PAD = NW * EPW
PAIRS = CHUNKS // 2

F32 = jnp.float32


def _make_norm():
  mesh = plsc.VectorSubcoreMesh(core_axis_name="c", subcore_axis_name="s")

  @functools.partial(
      pl.kernel,
      out_type=jax.ShapeDtypeStruct((NW * CHUNKS, K), F32),
      mesh=mesh,
      compiler_params=pltpu.CompilerParams(use_tc_tiling_on_sc=False,
                                           needs_layout_passes=False),
      scratch_types=[
          pltpu.VMEM((CHUNKS, K), jnp.int32),
          pltpu.VMEM((CHUNKS, K), jnp.int32),
          pltpu.VMEM((CHUNKS, K), F32),
          pltpu.VMEM((N,), F32),
      ],
  )
  def normk(src_hbm, dst_hbm, ew_hbm, dinv_hbm, out_hbm,
            src_v, dst_v, ew_v, dinv_v):
    c = lax.axis_index("c")
    s = lax.axis_index("s")
    base = (c * NS + s) * CHUNKS
    pltpu.sync_copy(src_hbm.at[pl.ds(base, CHUNKS)], src_v)
    pltpu.sync_copy(dst_hbm.at[pl.ds(base, CHUNKS)], dst_v)
    pltpu.sync_copy(ew_hbm.at[pl.ds(base, CHUNKS)], ew_v)
    pltpu.sync_copy(dinv_hbm, dinv_v)

    def nchunk(g, _):
      for j in range(K // L):
        sl = pl.ds(j * L, L)
        s16 = src_v[g, sl]
        d16 = dst_v[g, sl]
        w16 = ew_v[g, sl]
        nv = plsc.load_gather(dinv_v, [s16]) * w16
        ew_v[g, sl] = nv * plsc.load_gather(dinv_v, [d16])
      return 0
    lax.fori_loop(0, CHUNKS, nchunk, 0)
    pltpu.sync_copy(ew_v, out_hbm.at[pl.ds(base, CHUNKS)])

  return normk


def _make_agg(w):
  rows_per_sub = NP // NS
  zcopies = rows_per_sub // K
  mesh = plsc.VectorSubcoreMesh(core_axis_name="c", subcore_axis_name="s")

  @functools.partial(
      pl.kernel,
      out_type=jax.ShapeDtypeStruct((NC, NP, w), F32),
      mesh=mesh,
      compiler_params=pltpu.CompilerParams(use_tc_tiling_on_sc=False,
                                           needs_layout_passes=False),
      scratch_types=[
          pltpu.VMEM((CHUNKS, K), jnp.int32),
          pltpu.VMEM((CHUNKS, K), jnp.int32),
          pltpu.VMEM((CHUNKS, K), F32),
          pltpu.VMEM((K, w), F32),
          pltpu.VMEM((K, w), F32),
          pltpu.VMEM_SHARED((NP, w), F32),
          pltpu.SemaphoreType.DMA,
          pltpu.SemaphoreType.DMA,
          pltpu.SemaphoreType.DMA,
          pltpu.SemaphoreType.DMA,
      ],
  )
  def agg(g_hbm, src_hbm, dst_hbm, norm_hbm, out_hbm,
          src_v, dst_v, norm_v, rows_a, rows_b, acc_sh,
          sem_ga, sem_gb, sem_sa, sem_sb):
    c = lax.axis_index("c")
    s = lax.axis_index("s")
    base = (c * NS + s) * CHUNKS

    def g_start(g, rows, sem):
      pltpu.async_copy(g_hbm.at[src_v.at[g]], rows, sem)

    def g_wait(g, rows, sem):
      pltpu.make_async_copy(g_hbm.at[src_v.at[g]], rows, sem).wait()

    def s_start(g, rows, sem):
      pltpu.async_copy(rows, acc_sh.at[dst_v.at[g]], sem, add=True)

    def s_wait(g, rows, sem):
      pltpu.make_async_copy(rows, acc_sh.at[dst_v.at[g]], sem).wait()

    pltpu.sync_copy(src_hbm.at[pl.ds(base, CHUNKS)], src_v)
    g_start(0, rows_a, sem_ga)
    pltpu.sync_copy(dst_hbm.at[pl.ds(base, CHUNKS)], dst_v)
    pltpu.sync_copy(norm_hbm.at[pl.ds(base, CHUNKS)], norm_v)

    def zrow(i, _):
      for j in range(w // L):
        rows_b[i, pl.ds(j * L, L)] = jnp.zeros((L,), F32)
      return 0
    lax.fori_loop(0, K, zrow, 0)
    for i in range(zcopies):
      pltpu.async_copy(rows_b, acc_sh.at[pl.ds(s * rows_per_sub + i * K, K)],
                       sem_sa)
    for i in range(zcopies):
      pltpu.make_async_copy(
          rows_b, acc_sh.at[pl.ds(s * rows_per_sub + i * K, K)],
          sem_sa).wait()
    plsc.subcore_barrier()

    def scale(g, rows):
      def grp(j, _):
        nv = norm_v[g, pl.ds(j * L, L)]
        for t in range(L):
          nt = nv[t]
          e = j * L + t
          for jj in range(w // L):
            sl = pl.ds(jj * L, L)
            rows[e, sl] = rows[e, sl] * nt
        return 0
      lax.fori_loop(0, K // L, grp, 0, unroll=2)

    s_start(0, rows_b, sem_sb)

    def pair(gi, _):
      ga = gi * 2
      gb = ga + 1
      s_wait(gb, rows_b, sem_sb)
      g_start(gb, rows_b, sem_gb)
      g_wait(ga, rows_a, sem_ga)
      scale(ga, rows_a)
      s_start(ga, rows_a, sem_sa)
      g_wait(gb, rows_b, sem_gb)
      scale(gb, rows_b)
      s_wait(ga, rows_a, sem_sa)
      g_start(jnp.minimum(ga + 2, CHUNKS - 1), rows_a, sem_ga)
      s_start(gb, rows_b, sem_sb)
      return 0
    lax.fori_loop(0, PAIRS, pair, 0)

    g_wait(CHUNKS - 1, rows_a, sem_ga)
    s_wait(CHUNKS - 1, rows_b, sem_sb)

    plsc.subcore_barrier()
    for i in range(zcopies):
      sl = pl.ds(s * rows_per_sub + i * K, K)
      pltpu.async_copy(acc_sh.at[sl], out_hbm.at[c, sl], sem_sa)
    for i in range(zcopies):
      sl = pl.ds(s * rows_per_sub + i * K, K)
      pltpu.make_async_copy(acc_sh.at[sl], out_hbm.at[c, sl], sem_sa).wait()

  return agg


CH2 = 2 * CHUNKS
PAIRS2 = CH2 // 2


def _make_agg128():
  w = 64
  rows_per_sub = NP // NS
  zcopies = rows_per_sub // K
  mesh = plsc.VectorSubcoreMesh(core_axis_name="c", subcore_axis_name="s")

  @functools.partial(
      pl.kernel,
      out_type=jax.ShapeDtypeStruct((NC, NP, w), F32),
      mesh=mesh,
      compiler_params=pltpu.CompilerParams(use_tc_tiling_on_sc=False,
                                           needs_layout_passes=False),
      scratch_types=[
          pltpu.VMEM((CH2, K), jnp.int32),
          pltpu.VMEM((CH2, K), jnp.int32),
          pltpu.VMEM((CH2, K), F32),
          pltpu.VMEM((K, w), F32),
          pltpu.VMEM((K, w), F32),
          pltpu.VMEM_SHARED((NP, w), F32),
          pltpu.SemaphoreType.DMA,
          pltpu.SemaphoreType.DMA,
          pltpu.SemaphoreType.DMA,
          pltpu.SemaphoreType.DMA,
      ],
  )
  def agg(g_hbm, src_hbm, dst_hbm, norm_hbm, out_hbm,
          src_v, dst_v, norm_v, rows_a, rows_b, acc_sh,
          sem_ga, sem_gb, sem_sa, sem_sb):
    c = lax.axis_index("c")
    s = lax.axis_index("s")
    base = s * CH2

    def g_start(g, rows, sem):
      pltpu.async_copy(g_hbm.at[c].at[src_v.at[g]], rows, sem)

    def g_wait(g, rows, sem):
      pltpu.make_async_copy(g_hbm.at[c].at[src_v.at[g]], rows, sem).wait()

    def s_start(g, rows, sem):
      pltpu.async_copy(rows, acc_sh.at[dst_v.at[g]], sem, add=True)

    def s_wait(g, rows, sem):
      pltpu.make_async_copy(rows, acc_sh.at[dst_v.at[g]], sem).wait()

    pltpu.sync_copy(src_hbm.at[pl.ds(base, CH2)], src_v)
    g_start(0, rows_a, sem_ga)
    pltpu.sync_copy(dst_hbm.at[pl.ds(base, CH2)], dst_v)
    pltpu.sync_copy(norm_hbm.at[pl.ds(base, CH2)], norm_v)

    def zrow(i, _):
      for j in range(w // L):
        rows_b[i, pl.ds(j * L, L)] = jnp.zeros((L,), F32)
      return 0
    lax.fori_loop(0, K, zrow, 0)
    for i in range(zcopies):
      pltpu.async_copy(rows_b, acc_sh.at[pl.ds(s * rows_per_sub + i * K, K)],
                       sem_sa)
    for i in range(zcopies):
      pltpu.make_async_copy(
          rows_b, acc_sh.at[pl.ds(s * rows_per_sub + i * K, K)],
          sem_sa).wait()
    plsc.subcore_barrier()

    def scale(g, rows):
      def grp(j, _):
        nv = norm_v[g, pl.ds(j * L, L)]
        for t in range(L):
          nt = nv[t]
          e = j * L + t
          for jj in range(w // L):
            sl = pl.ds(jj * L, L)
            rows[e, sl] = rows[e, sl] * nt
        return 0
      lax.fori_loop(0, K // L, grp, 0, unroll=2)

    s_start(0, rows_b, sem_sb)

    def pair(gi, _):
      ga = gi * 2
      gb = ga + 1
      s_wait(gb, rows_b, sem_sb)
      g_start(gb, rows_b, sem_gb)
      g_wait(ga, rows_a, sem_ga)
      scale(ga, rows_a)
      s_start(ga, rows_a, sem_sa)
      g_wait(gb, rows_b, sem_gb)
      scale(gb, rows_b)
      s_wait(ga, rows_a, sem_sa)
      g_start(jnp.minimum(ga + 2, CH2 - 1), rows_a, sem_ga)
      s_start(gb, rows_b, sem_sb)
      return 0
    lax.fori_loop(0, PAIRS2, pair, 0)

    g_wait(CH2 - 1, rows_a, sem_ga)
    s_wait(CH2 - 1, rows_b, sem_sb)

    plsc.subcore_barrier()
    for i in range(zcopies):
      sl = pl.ds(s * rows_per_sub + i * K, K)
      pltpu.async_copy(acc_sh.at[sl], out_hbm.at[c, sl], sem_sa)
    for i in range(zcopies):
      sl = pl.ds(s * rows_per_sub + i * K, K)
      pltpu.make_async_copy(acc_sh.at[sl], out_hbm.at[c, sl], sem_sa).wait()

  return agg


def _bn(h, g, b):
  mu = jnp.mean(h, axis=0, keepdims=True)
  var = jnp.var(h, axis=0, keepdims=True)
  return g * (h - mu) / jnp.sqrt(var + 1e-5) + b


def _tc(fn, out_shape, *args):
  return pl.pallas_call(fn, out_shape=out_shape)(*args)


DIMS = [(128, 128), (128, 128), (128, 64), (64, 32), (32, 64), (64, 128),
        (128, 64), (64, 32)]
AGG_W = [do for _, do in DIMS]


def kernel(x, edge_index, edge_weight, params):
  p = params
  src = edge_index[0]
  dst = edge_index[1]
  pad = E_PAD - E
  srcp = jnp.pad(src, (0, pad)).reshape(NW * CHUNKS, K)
  dstp = jnp.pad(dst, (0, pad)).reshape(NW * CHUNKS, K)
  ewp = jnp.pad(edge_weight, (0, pad)).reshape(NW * CHUNKS, K)

  aggs = {w: _make_agg(w) for w in (16, 32, 64)}
  agg128 = _make_agg128()
  normk = _make_norm()

  ones16 = jnp.ones((N, 16), F32)
  s0 = aggs[16](ones16, srcp, dstp, ewp)

  def tc0(x_ref, w1_ref, g_ref, be_ref, s0_ref, dinv_ref, dinv2_ref, a1_ref):
    deg = (s0_ref[0, :N, 0:1] + s0_ref[1, :N, 0:1]) + 1.0
    dinv = lax.rsqrt(deg)
    dinv_ref[...] = dinv
    dinv2_ref[...] = dinv * dinv
    h = _bn(x_ref[...], g_ref[...], be_ref[...])
    a1 = jnp.dot(h, w1_ref[...], preferred_element_type=F32)
    a1_ref[0, :, :] = a1[:, :64]
    a1_ref[1, :, :] = a1[:, 64:]

  dinv, dinv2, a = _tc(
      tc0,
      (jax.ShapeDtypeStruct((N, 1), F32),
       jax.ShapeDtypeStruct((N, 1), F32),
       jax.ShapeDtypeStruct((2, N, 64), F32)),
      x, p["W1"], p["g1"], p["be1"], s0)

  normp = normk(srcp, dstp, ewp, dinv.reshape((N,)))

  for i in range(1, 9):
    wi = AGG_W[i - 1]
    if wi == 128:
      s_i = agg128(a, srcp, dstp, normp)
    else:
      s_i = aggs[wi](a, srcp, dstp, normp)
    wi128 = wi == 128

    if i < 8:
      wn = AGG_W[i]
      wn128 = wn == 128
      out_shape = (jax.ShapeDtypeStruct((2, N, 64), F32) if wn128
                   else jax.ShapeDtypeStruct((N, wn), F32))

      def tci(s_ref, a_ref, dinv2_ref, b_ref, g_ref, be_ref, wn_ref, an_ref,
              wi128=wi128, wn128=wn128):
        if wi128:
          s_cat = jnp.concatenate([s_ref[0, :N, :], s_ref[1, :N, :]], axis=1)
          a_cat = jnp.concatenate([a_ref[0], a_ref[1]], axis=1)
        else:
          s_cat = s_ref[0, :N, :] + s_ref[1, :N, :]
          a_cat = a_ref[...]
        conv = s_cat + a_cat * dinv2_ref[...]
        r = conv + b_ref[...]
        h = jax.nn.relu(_bn(r, g_ref[...], be_ref[...]))
        an = jnp.dot(h, wn_ref[...], preferred_element_type=F32)
        if wn128:
          an_ref[0, :, :] = an[:, :64]
          an_ref[1, :, :] = an[:, 64:]
        else:
          an_ref[...] = an

      a = _tc(
          tci, out_shape,
          s_i, a, dinv2, p["b%d" % i],
          p["g%d" % (i + 1)], p["be%d" % (i + 1)], p["W%d" % (i + 1)])
    else:
      def tc8(s_ref, a_ref, dinv2_ref, b_ref, g9_ref, be9_ref,
              lw1_ref, lb1_ref, g10_ref, be10_ref, lw2_ref, lb2_ref,
              out_ref):
        s_cat = s_ref[0, :N, :] + s_ref[1, :N, :]
        conv = s_cat + a_ref[...] * dinv2_ref[...]
        h = _bn(conv + b_ref[...], g9_ref[...], be9_ref[...])
        t1 = jnp.dot(jax.nn.relu(h), lw1_ref[...],
                     preferred_element_type=F32) + lb1_ref[...]
        t1 = _bn(t1, g10_ref[...], be10_ref[...])
        out_ref[...] = jnp.dot(jax.nn.relu(t1), lw2_ref[...],
                               preferred_element_type=F32) + lb2_ref[...]

      out = _tc(
          tc8, jax.ShapeDtypeStruct((N, 40), F32),
          s_i, a, dinv2, p["b8"], p["g9"], p["be9"],
          p["lw1"], p["lb1"], p["g10"], p["be10"], p["lw2"], p["lb2"])

  return out


# speedup vs baseline: 1.5355x; 1.0289x over previous
import functools

import jax
import jax.numpy as jnp
from jax import lax
from jax.experimental import pallas as pl
from jax.experimental.pallas import tpu as pltpu
from jax.experimental.pallas import tpu_sc as plsc

N = 10000
NP = 10240
E = 320000
NC, NS, L = 2, 16, 16
NW = NC * NS
K = 128
CHUNKS = 80
EPW = CHUNKS * K
E_PAD = NW * EPW
PAIRS = CHUNKS // 2

F32 = jnp.float32


def _make_norm():
  mesh = plsc.VectorSubcoreMesh(core_axis_name="c", subcore_axis_name="s")

  @functools.partial(
      pl.kernel,
      out_type=jax.ShapeDtypeStruct((NW * CHUNKS, K), F32),
      mesh=mesh,
      compiler_params=pltpu.CompilerParams(use_tc_tiling_on_sc=False,
                                           needs_layout_passes=False),
      scratch_types=[
          pltpu.VMEM((CHUNKS, K), jnp.int32),
          pltpu.VMEM((CHUNKS, K), jnp.int32),
          pltpu.VMEM((CHUNKS, K), F32),
          pltpu.VMEM((N,), F32),
      ],
  )
  def normk(src_hbm, dst_hbm, ew_hbm, dinv_hbm, out_hbm,
            src_v, dst_v, ew_v, dinv_v):
    c = lax.axis_index("c")
    s = lax.axis_index("s")
    base = (c * NS + s) * CHUNKS
    pltpu.sync_copy(src_hbm.at[pl.ds(base, CHUNKS)], src_v)
    pltpu.sync_copy(dst_hbm.at[pl.ds(base, CHUNKS)], dst_v)
    pltpu.sync_copy(ew_hbm.at[pl.ds(base, CHUNKS)], ew_v)
    pltpu.sync_copy(dinv_hbm, dinv_v)

    def nchunk(g, _):
      for j in range(K // L):
        sl = pl.ds(j * L, L)
        s16 = src_v[g, sl]
        d16 = dst_v[g, sl]
        w16 = ew_v[g, sl]
        nv = plsc.load_gather(dinv_v, [s16]) * w16
        ew_v[g, sl] = nv * plsc.load_gather(dinv_v, [d16])
      return 0
    lax.fori_loop(0, CHUNKS, nchunk, 0)
    pltpu.sync_copy(ew_v, out_hbm.at[pl.ds(base, CHUNKS)])

  return normk


def _make_agg(w):
  rows_per_sub = NP // NS
  zcopies = rows_per_sub // K
  mesh = plsc.VectorSubcoreMesh(core_axis_name="c", subcore_axis_name="s")

  @functools.partial(
      pl.kernel,
      out_type=jax.ShapeDtypeStruct((NC, NP, w), F32),
      mesh=mesh,
      compiler_params=pltpu.CompilerParams(use_tc_tiling_on_sc=False,
                                           needs_layout_passes=False),
      scratch_types=[
          pltpu.VMEM((CHUNKS, K), jnp.int32),
          pltpu.VMEM((CHUNKS, K), jnp.int32),
          pltpu.VMEM((CHUNKS, K), F32),
          pltpu.VMEM((K, w), F32),
          pltpu.VMEM((K, w), F32),
          pltpu.VMEM_SHARED((NP, w), F32),
          pltpu.SemaphoreType.DMA,
          pltpu.SemaphoreType.DMA,
          pltpu.SemaphoreType.DMA,
          pltpu.SemaphoreType.DMA,
      ],
  )
  def agg(g_hbm, src_hbm, dst_hbm, norm_hbm, out_hbm,
          src_v, dst_v, norm_v, rows_a, rows_b, acc_sh,
          sem_ga, sem_gb, sem_sa, sem_sb):
    c = lax.axis_index("c")
    s = lax.axis_index("s")
    base = (c * NS + s) * CHUNKS

    def g_start(g, rows, sem):
      pltpu.async_copy(g_hbm.at[src_v.at[g]], rows, sem)

    def g_wait(g, rows, sem):
      pltpu.make_async_copy(g_hbm.at[src_v.at[g]], rows, sem).wait()

    def s_start(g, rows, sem):
      pltpu.async_copy(rows, acc_sh.at[dst_v.at[g]], sem, add=True)

    def s_wait(g, rows, sem):
      pltpu.make_async_copy(rows, acc_sh.at[dst_v.at[g]], sem).wait()

    pltpu.sync_copy(src_hbm.at[pl.ds(base, CHUNKS)], src_v)
    g_start(0, rows_a, sem_ga)
    pltpu.sync_copy(dst_hbm.at[pl.ds(base, CHUNKS)], dst_v)
    pltpu.sync_copy(norm_hbm.at[pl.ds(base, CHUNKS)], norm_v)

    def zrow(i, _):
      for j in range(w // L):
        rows_b[i, pl.ds(j * L, L)] = jnp.zeros((L,), F32)
      return 0
    lax.fori_loop(0, K, zrow, 0)
    for i in range(zcopies):
      pltpu.async_copy(rows_b, acc_sh.at[pl.ds(s * rows_per_sub + i * K, K)],
                       sem_sa)
    for i in range(zcopies):
      pltpu.make_async_copy(
          rows_b, acc_sh.at[pl.ds(s * rows_per_sub + i * K, K)],
          sem_sa).wait()
    plsc.subcore_barrier()

    def scale(g, rows):
      def grp(j, _):
        nv = norm_v[g, pl.ds(j * L, L)]
        for t in range(L):
          nt = nv[t]
          e = j * L + t
          for jj in range(w // L):
            sl = pl.ds(jj * L, L)
            rows[e, sl] = rows[e, sl] * nt
        return 0
      lax.fori_loop(0, K // L, grp, 0, unroll=4)

    s_start(0, rows_b, sem_sb)

    def pair(gi, _):
      ga = gi * 2
      gb = ga + 1
      s_wait(gb, rows_b, sem_sb)
      g_start(gb, rows_b, sem_gb)
      g_wait(ga, rows_a, sem_ga)
      scale(ga, rows_a)
      s_start(ga, rows_a, sem_sa)
      g_wait(gb, rows_b, sem_gb)
      scale(gb, rows_b)
      s_wait(ga, rows_a, sem_sa)
      g_start(jnp.minimum(ga + 2, CHUNKS - 1), rows_a, sem_ga)
      s_start(gb, rows_b, sem_sb)
      return 0
    lax.fori_loop(0, PAIRS, pair, 0)

    g_wait(CHUNKS - 1, rows_a, sem_ga)
    s_wait(CHUNKS - 1, rows_b, sem_sb)

    plsc.subcore_barrier()
    for i in range(zcopies):
      sl = pl.ds(s * rows_per_sub + i * K, K)
      pltpu.async_copy(acc_sh.at[sl], out_hbm.at[c, sl], sem_sa)
    for i in range(zcopies):
      sl = pl.ds(s * rows_per_sub + i * K, K)
      pltpu.make_async_copy(acc_sh.at[sl], out_hbm.at[c, sl], sem_sa).wait()

  return agg


CH2 = 2 * CHUNKS
PAIRS2 = CH2 // 2


def _make_agg128():
  w = 64
  rows_per_sub = NP // NS
  zcopies = rows_per_sub // K
  mesh = plsc.VectorSubcoreMesh(core_axis_name="c", subcore_axis_name="s")

  @functools.partial(
      pl.kernel,
      out_type=jax.ShapeDtypeStruct((NC, NP, w), F32),
      mesh=mesh,
      compiler_params=pltpu.CompilerParams(use_tc_tiling_on_sc=False,
                                           needs_layout_passes=False),
      scratch_types=[
          pltpu.VMEM((CH2, K), jnp.int32),
          pltpu.VMEM((CH2, K), jnp.int32),
          pltpu.VMEM((CH2, K), F32),
          pltpu.VMEM((K, w), F32),
          pltpu.VMEM((K, w), F32),
          pltpu.VMEM_SHARED((NP, w), F32),
          pltpu.SemaphoreType.DMA,
          pltpu.SemaphoreType.DMA,
          pltpu.SemaphoreType.DMA,
          pltpu.SemaphoreType.DMA,
      ],
  )
  def agg(g_hbm, src_hbm, dst_hbm, norm_hbm, out_hbm,
          src_v, dst_v, norm_v, rows_a, rows_b, acc_sh,
          sem_ga, sem_gb, sem_sa, sem_sb):
    c = lax.axis_index("c")
    s = lax.axis_index("s")
    base = s * CH2

    def g_start(g, rows, sem):
      pltpu.async_copy(g_hbm.at[c].at[src_v.at[g]], rows, sem)

    def g_wait(g, rows, sem):
      pltpu.make_async_copy(g_hbm.at[c].at[src_v.at[g]], rows, sem).wait()

    def s_start(g, rows, sem):
      pltpu.async_copy(rows, acc_sh.at[dst_v.at[g]], sem, add=True)

    def s_wait(g, rows, sem):
      pltpu.make_async_copy(rows, acc_sh.at[dst_v.at[g]], sem).wait()

    pltpu.sync_copy(src_hbm.at[pl.ds(base, CH2)], src_v)
    g_start(0, rows_a, sem_ga)
    pltpu.sync_copy(dst_hbm.at[pl.ds(base, CH2)], dst_v)
    pltpu.sync_copy(norm_hbm.at[pl.ds(base, CH2)], norm_v)

    def zrow(i, _):
      for j in range(w // L):
        rows_b[i, pl.ds(j * L, L)] = jnp.zeros((L,), F32)
      return 0
    lax.fori_loop(0, K, zrow, 0)
    for i in range(zcopies):
      pltpu.async_copy(rows_b, acc_sh.at[pl.ds(s * rows_per_sub + i * K, K)],
                       sem_sa)
    for i in range(zcopies):
      pltpu.make_async_copy(
          rows_b, acc_sh.at[pl.ds(s * rows_per_sub + i * K, K)],
          sem_sa).wait()
    plsc.subcore_barrier()

    def scale(g, rows):
      def grp(j, _):
        nv = norm_v[g, pl.ds(j * L, L)]
        for t in range(L):
          nt = nv[t]
          e = j * L + t
          for jj in range(w // L):
            sl = pl.ds(jj * L, L)
            rows[e, sl] = rows[e, sl] * nt
        return 0
      lax.fori_loop(0, K // L, grp, 0, unroll=4)

    s_start(0, rows_b, sem_sb)

    def pair(gi, _):
      ga = gi * 2
      gb = ga + 1
      s_wait(gb, rows_b, sem_sb)
      g_start(gb, rows_b, sem_gb)
      g_wait(ga, rows_a, sem_ga)
      scale(ga, rows_a)
      s_start(ga, rows_a, sem_sa)
      g_wait(gb, rows_b, sem_gb)
      scale(gb, rows_b)
      s_wait(ga, rows_a, sem_sa)
      g_start(jnp.minimum(ga + 2, CH2 - 1), rows_a, sem_ga)
      s_start(gb, rows_b, sem_sb)
      return 0
    lax.fori_loop(0, PAIRS2, pair, 0)

    g_wait(CH2 - 1, rows_a, sem_ga)
    s_wait(CH2 - 1, rows_b, sem_sb)

    plsc.subcore_barrier()
    for i in range(zcopies):
      sl = pl.ds(s * rows_per_sub + i * K, K)
      pltpu.async_copy(acc_sh.at[sl], out_hbm.at[c, sl], sem_sa)
    for i in range(zcopies):
      sl = pl.ds(s * rows_per_sub + i * K, K)
      pltpu.make_async_copy(acc_sh.at[sl], out_hbm.at[c, sl], sem_sa).wait()

  return agg


def _bn(h, g, b):
  mu = jnp.mean(h, axis=0, keepdims=True)
  var = jnp.var(h, axis=0, keepdims=True)
  return g * (h - mu) / jnp.sqrt(var + 1e-5) + b


def _tc(fn, out_shape, *args):
  return pl.pallas_call(fn, out_shape=out_shape)(*args)


DIMS = [(128, 128), (128, 128), (128, 64), (64, 32), (32, 64), (64, 128),
        (128, 64), (64, 32)]
AGG_W = [do for _, do in DIMS]


def kernel(x, edge_index, edge_weight, params):
  p = params
  src = edge_index[0]
  dst = edge_index[1]
  pad = E_PAD - E
  srcp = jnp.pad(src, (0, pad)).reshape(NW * CHUNKS, K)
  dstp = jnp.pad(dst, (0, pad)).reshape(NW * CHUNKS, K)
  ewp = jnp.pad(edge_weight, (0, pad)).reshape(NW * CHUNKS, K)

  aggs = {w: _make_agg(w) for w in (16, 32, 64)}
  agg128 = _make_agg128()
  normk = _make_norm()

  ones16 = jnp.ones((N, 16), F32)
  s0 = aggs[16](ones16, srcp, dstp, ewp)

  def tc0(x_ref, w1_ref, g_ref, be_ref, s0_ref, dinv_ref, dinv2_ref, a1_ref):
    deg = (s0_ref[0, :N, 0:1] + s0_ref[1, :N, 0:1]) + 1.0
    dinv = lax.rsqrt(deg)
    dinv_ref[...] = dinv
    dinv2_ref[...] = dinv * dinv
    h = _bn(x_ref[...], g_ref[...], be_ref[...])
    a1 = jnp.dot(h, w1_ref[...], preferred_element_type=F32)
    a1_ref[0, :, :] = a1[:, :64]
    a1_ref[1, :, :] = a1[:, 64:]

  dinv, dinv2, a = _tc(
      tc0,
      (jax.ShapeDtypeStruct((N, 1), F32),
       jax.ShapeDtypeStruct((N, 1), F32),
       jax.ShapeDtypeStruct((2, N, 64), F32)),
      x, p["W1"], p["g1"], p["be1"], s0)

  normp = normk(srcp, dstp, ewp, dinv.reshape((N,)))

  for i in range(1, 9):
    wi = AGG_W[i - 1]
    if wi == 128:
      s_i = agg128(a, srcp, dstp, normp)
    else:
      s_i = aggs[wi](a, srcp, dstp, normp)
    wi128 = wi == 128

    if i < 8:
      wn = AGG_W[i]
      wn128 = wn == 128
      out_shape = (jax.ShapeDtypeStruct((2, N, 64), F32) if wn128
                   else jax.ShapeDtypeStruct((N, wn), F32))

      def tci(s_ref, a_ref, dinv2_ref, b_ref, g_ref, be_ref, wn_ref, an_ref,
              wi128=wi128, wn128=wn128):
        if wi128:
          s_cat = jnp.concatenate([s_ref[0, :N, :], s_ref[1, :N, :]], axis=1)
          a_cat = jnp.concatenate([a_ref[0], a_ref[1]], axis=1)
        else:
          s_cat = s_ref[0, :N, :] + s_ref[1, :N, :]
          a_cat = a_ref[...]
        conv = s_cat + a_cat * dinv2_ref[...]
        r = conv + b_ref[...]
        h = jax.nn.relu(_bn(r, g_ref[...], be_ref[...]))
        an = jnp.dot(h, wn_ref[...], preferred_element_type=F32)
        if wn128:
          an_ref[0, :, :] = an[:, :64]
          an_ref[1, :, :] = an[:, 64:]
        else:
          an_ref[...] = an

      a = _tc(
          tci, out_shape,
          s_i, a, dinv2, p["b%d" % i],
          p["g%d" % (i + 1)], p["be%d" % (i + 1)], p["W%d" % (i + 1)])
    else:
      def tc8(s_ref, a_ref, dinv2_ref, b_ref, g9_ref, be9_ref,
              lw1_ref, lb1_ref, g10_ref, be10_ref, lw2_ref, lb2_ref,
              out_ref):
        s_cat = s_ref[0, :N, :] + s_ref[1, :N, :]
        conv = s_cat + a_ref[...] * dinv2_ref[...]
        h = _bn(conv + b_ref[...], g9_ref[...], be9_ref[...])
        t1 = jnp.dot(jax.nn.relu(h), lw1_ref[...],
                     preferred_element_type=F32) + lb1_ref[...]
        t1 = _bn(t1, g10_ref[...], be10_ref[...])
        out_ref[...] = jnp.dot(jax.nn.relu(t1), lw2_ref[...],
                               preferred_element_type=F32) + lb2_ref[...]

      out = _tc(
          tc8, jax.ShapeDtypeStruct((N, 40), F32),
          s_i, a, dinv2, p["b8"], p["g9"], p["be9"],
          p["lw1"], p["lb1"], p["g10"], p["be10"], p["lw2"], p["lb2"])

  return out
